# baseline (speedup 1.0000x reference)
import jax
import jax.numpy as jnp
from jax.experimental import pallas as pl
from jax.experimental.pallas import tpu as pltpu

_NCHUNK = 8


def _gcn_kernel(ehr_hbm, ddi_hbm, w1_ref, b1_ref, w2_ref, b2_ref, w3_ref,
                b3_ref, oehr_hbm, oddi_hbm, adj32, adj_bf, h_scr, s_scr,
                out_buf, in_sems, out_sems):
    b = pl.program_id(0)
    v = adj32.shape[0]
    ch = v // _NCHUNK
    f32 = jnp.float32
    bf16 = jnp.bfloat16

    for c in range(_NCHUNK):
        rows = pl.ds(c * ch, ch)

        @pl.when(b == 0)
        def _(rows=rows, c=c):
            pltpu.make_async_copy(ehr_hbm.at[rows], adj32.at[rows],
                                  in_sems.at[c]).start()

        @pl.when(b == 1)
        def _(rows=rows, c=c):
            pltpu.make_async_copy(ddi_hbm.at[rows], adj32.at[rows],
                                  in_sems.at[c]).start()

    w1b = w1_ref[...].astype(bf16)
    b1v = b1_ref[...]
    wout = jnp.where(b == 0, w2_ref[...], w3_ref[...]).astype(bf16)
    bout = jnp.where(b == 0, b2_ref[...], b3_ref[...])
    hv = v // 2

    for c in range(_NCHUNK):
        rows = pl.ds(c * ch, ch)
        pltpu.make_async_copy(adj32.at[rows], adj32.at[rows],
                              in_sems.at[c]).wait()
        ab = adj32[rows, :].astype(bf16)
        adj_bf[rows, :] = ab
        hc = jnp.dot(ab, w1b, preferred_element_type=f32) + b1v
        h_scr[rows, :] = jnp.maximum(hc, 0.0).astype(bf16)

        if c == _NCHUNK // 2 - 1:
            s_scr[pl.ds(0, hv), :] = jnp.dot(
                h_scr[pl.ds(0, hv), :], wout,
                preferred_element_type=f32).astype(bf16)
        if c >= _NCHUNK // 2:
            for r in (2 * (c - _NCHUNK // 2), 2 * (c - _NCHUNK // 2) + 1):
                rrows = pl.ds(r * ch, ch)
                out_buf[rrows, :] = jnp.dot(
                    adj_bf[rrows, pl.ds(0, hv)], s_scr[pl.ds(0, hv), :],
                    preferred_element_type=f32) + bout

    s_scr[pl.ds(hv, hv), :] = jnp.dot(
        h_scr[pl.ds(hv, hv), :], wout, preferred_element_type=f32).astype(bf16)

    for c in range(_NCHUNK):
        rows = pl.ds(c * ch, ch)
        oc = jnp.dot(adj_bf[rows, pl.ds(hv, hv)], s_scr[pl.ds(hv, hv), :],
                     preferred_element_type=f32)
        out_buf[rows, :] = out_buf[rows, :] + oc

        @pl.when(b == 0)
        def _(rows=rows, c=c):
            pltpu.make_async_copy(out_buf.at[rows], oehr_hbm.at[rows],
                                  out_sems.at[c]).start()

        @pl.when(b == 1)
        def _(rows=rows, c=c):
            pltpu.make_async_copy(out_buf.at[rows], oddi_hbm.at[rows],
                                  out_sems.at[c]).start()

    for c in range(_NCHUNK):
        rows = pl.ds(c * ch, ch)
        pltpu.make_async_copy(out_buf.at[rows], out_buf.at[rows],
                              out_sems.at[c]).wait()


def kernel(ehr_adj_norm, ddi_adj_norm, w1, b1, w2, b2, w3, b3):
    f32 = jnp.float32
    v = ehr_adj_norm.shape[0]
    e = w1.shape[1]
    assert v % (_NCHUNK * 8) == 0 and _NCHUNK % 2 == 0 and e % 128 == 0

    b1r = b1.reshape(1, e)
    b2r = b2.reshape(1, e)
    b3r = b3.reshape(1, e)

    out_ehr, out_ddi = pl.pallas_call(
        _gcn_kernel,
        out_shape=(jax.ShapeDtypeStruct((v, e), f32),
                   jax.ShapeDtypeStruct((v, e), f32)),
        grid=(2,),
        in_specs=[
            pl.BlockSpec(memory_space=pl.ANY),
            pl.BlockSpec(memory_space=pl.ANY),
            pl.BlockSpec((v, e), lambda b: (0, 0)),
            pl.BlockSpec((1, e), lambda b: (0, 0)),
            pl.BlockSpec((e, e), lambda b: (0, 0)),
            pl.BlockSpec((1, e), lambda b: (0, 0)),
            pl.BlockSpec((e, e), lambda b: (0, 0)),
            pl.BlockSpec((1, e), lambda b: (0, 0)),
        ],
        out_specs=(pl.BlockSpec(memory_space=pl.ANY),
                   pl.BlockSpec(memory_space=pl.ANY)),
        scratch_shapes=[
            pltpu.VMEM((v, v), f32),
            pltpu.VMEM((v, v), jnp.bfloat16),
            pltpu.VMEM((v, e), jnp.bfloat16),
            pltpu.VMEM((v, e), jnp.bfloat16),
            pltpu.VMEM((v, e), f32),
            pltpu.SemaphoreType.DMA((_NCHUNK,)),
            pltpu.SemaphoreType.DMA((_NCHUNK,)),
        ],
        compiler_params=pltpu.CompilerParams(
            dimension_semantics=("parallel",)),
    )(ehr_adj_norm, ddi_adj_norm, w1, b1r, w2, b2r, w3, b3r)

    return out_ehr, out_ddi


# speedup vs baseline: 1.0140x; 1.0140x over previous
import jax
import jax.numpy as jnp
from jax.experimental import pallas as pl
from jax.experimental.pallas import tpu as pltpu

_NCHUNK = 8


def _gcn_kernel(ehr_hbm, ddi_hbm, w1_ref, b1_ref, w2_ref, b2_ref, w3_ref,
                b3_ref, oehr_hbm, oddi_hbm, adj32, adj_bf, h_scr, out_buf,
                in_sems, out_sems):
    b = pl.program_id(0)
    v = adj32.shape[0]
    ch = v // _NCHUNK
    f32 = jnp.float32
    bf16 = jnp.bfloat16

    for c in range(_NCHUNK):
        rows = pl.ds(c * ch, ch)

        @pl.when(b == 0)
        def _(rows=rows, c=c):
            pltpu.make_async_copy(ehr_hbm.at[rows], adj32.at[rows],
                                  in_sems.at[c]).start()

        @pl.when(b == 1)
        def _(rows=rows, c=c):
            pltpu.make_async_copy(ddi_hbm.at[rows], adj32.at[rows],
                                  in_sems.at[c]).start()

    w1b = w1_ref[...].astype(bf16)
    b1v = b1_ref[...]

    for c in range(_NCHUNK):
        rows = pl.ds(c * ch, ch)
        pltpu.make_async_copy(adj32.at[rows], adj32.at[rows],
                              in_sems.at[c]).wait()
        ab = adj32[rows, :].astype(bf16)
        adj_bf[rows, :] = ab
        hc = jnp.dot(ab, w1b, preferred_element_type=f32) + b1v
        h_scr[rows, :] = jnp.maximum(hc, 0.0).astype(bf16)

    wout = jnp.where(b == 0, w2_ref[...], w3_ref[...]).astype(bf16)
    bout = jnp.where(b == 0, b2_ref[...], b3_ref[...])
    s = jnp.dot(h_scr[...], wout, preferred_element_type=f32).astype(bf16)

    for c in range(_NCHUNK):
        rows = pl.ds(c * ch, ch)
        oc = jnp.dot(adj_bf[rows, :], s, preferred_element_type=f32) + bout
        out_buf[rows, :] = oc

        @pl.when(b == 0)
        def _(rows=rows, c=c):
            pltpu.make_async_copy(out_buf.at[rows], oehr_hbm.at[rows],
                                  out_sems.at[c]).start()

        @pl.when(b == 1)
        def _(rows=rows, c=c):
            pltpu.make_async_copy(out_buf.at[rows], oddi_hbm.at[rows],
                                  out_sems.at[c]).start()

    for c in range(_NCHUNK):
        rows = pl.ds(c * ch, ch)
        pltpu.make_async_copy(out_buf.at[rows], out_buf.at[rows],
                              out_sems.at[c]).wait()


def kernel(ehr_adj_norm, ddi_adj_norm, w1, b1, w2, b2, w3, b3):
    f32 = jnp.float32
    v = ehr_adj_norm.shape[0]
    e = w1.shape[1]
    assert v % (_NCHUNK * 8) == 0 and e % 128 == 0

    b1r = b1.reshape(1, e)
    b2r = b2.reshape(1, e)
    b3r = b3.reshape(1, e)

    out_ehr, out_ddi = pl.pallas_call(
        _gcn_kernel,
        out_shape=(jax.ShapeDtypeStruct((v, e), f32),
                   jax.ShapeDtypeStruct((v, e), f32)),
        grid=(2,),
        in_specs=[
            pl.BlockSpec(memory_space=pl.ANY),
            pl.BlockSpec(memory_space=pl.ANY),
            pl.BlockSpec((v, e), lambda b: (0, 0)),
            pl.BlockSpec((1, e), lambda b: (0, 0)),
            pl.BlockSpec((e, e), lambda b: (0, 0)),
            pl.BlockSpec((1, e), lambda b: (0, 0)),
            pl.BlockSpec((e, e), lambda b: (0, 0)),
            pl.BlockSpec((1, e), lambda b: (0, 0)),
        ],
        out_specs=(pl.BlockSpec(memory_space=pl.ANY),
                   pl.BlockSpec(memory_space=pl.ANY)),
        scratch_shapes=[
            pltpu.VMEM((v, v), f32),
            pltpu.VMEM((v, v), jnp.bfloat16),
            pltpu.VMEM((v, e), jnp.bfloat16),
            pltpu.VMEM((v, e), f32),
            pltpu.SemaphoreType.DMA((_NCHUNK,)),
            pltpu.SemaphoreType.DMA((_NCHUNK,)),
        ],
        compiler_params=pltpu.CompilerParams(
            dimension_semantics=("parallel",)),
    )(ehr_adj_norm, ddi_adj_norm, w1, b1r, w2, b2r, w3, b3r)

    return out_ehr, out_ddi
